# Initial kernel scaffold; baseline (speedup 1.0000x reference)
import functools

import jax
import jax.numpy as jnp
from jax import lax
from jax.experimental import pallas as pl
from jax.experimental.pallas import tpu as pltpu
from jax.experimental.pallas import tpu_sc as plsc

N_NODES = 10000
N_EDGES = 320000
NC = 2
NS = 16
NW = NC * NS
EPW = N_EDGES // NW
CH = 80
NCHUNK = EPW // CH
RPS = N_NODES // NS


def _sc_mesh():
    return plsc.VectorSubcoreMesh(core_axis_name="c", subcore_axis_name="s")


def _sc_degree_histogram(dst, zeros16, ones16):
    @functools.partial(
        pl.kernel,
        out_type=jax.ShapeDtypeStruct((NC, N_NODES, 16), jnp.float32),
        mesh=_sc_mesh(),
        scratch_types=[
            pltpu.VMEM((CH,), jnp.int32),
            pltpu.VMEM((CH, 16), jnp.float32),
            pltpu.VMEM_SHARED((N_NODES, 16), jnp.float32),
            pltpu.SemaphoreType.DMA,
        ],
    )
    def hist(dst_hbm, z_hbm, one_hbm, out_hbm, idx_v, ones_v, acc_sh, sem):
        c = lax.axis_index("c")
        s = lax.axis_index("s")
        row0 = s * RPS
        pltpu.sync_copy(one_hbm, ones_v)
        pltpu.sync_copy(z_hbm.at[pl.ds(row0, RPS)], acc_sh.at[pl.ds(row0, RPS)])
        plsc.subcore_barrier()
        base = (c * NS + s) * EPW

        @pl.loop(0, NCHUNK)
        def _(k):
            pltpu.sync_copy(dst_hbm.at[pl.ds(base + k * CH, CH)], idx_v)
            pltpu.sync_copy(ones_v, acc_sh.at[idx_v], add=True)

        plsc.subcore_barrier()
        pltpu.sync_copy(acc_sh.at[pl.ds(row0, RPS)],
                        out_hbm.at[c, pl.ds(row0, RPS)])

    return hist(dst, zeros16, ones16)


def _sc_scatter_rows(g, src, dst, zeros):
    d = g.shape[1]

    @functools.partial(
        pl.kernel,
        out_type=jax.ShapeDtypeStruct((NC, N_NODES, d), jnp.float32),
        mesh=_sc_mesh(),
        scratch_types=[
            pltpu.VMEM((CH,), jnp.int32),
            pltpu.VMEM((CH,), jnp.int32),
            pltpu.VMEM((CH, d), jnp.float32),
            pltpu.VMEM_SHARED((N_NODES, d), jnp.float32),
            pltpu.SemaphoreType.DMA,
        ],
    )
    def scat(g_hbm, src_hbm, dst_hbm, z_hbm, out_hbm, srcv, dstv, rows, acc_sh, sem):
        c = lax.axis_index("c")
        s = lax.axis_index("s")
        row0 = s * RPS
        pltpu.sync_copy(z_hbm.at[pl.ds(row0, RPS)], acc_sh.at[pl.ds(row0, RPS)])
        plsc.subcore_barrier()
        base = (c * NS + s) * EPW

        @pl.loop(0, NCHUNK)
        def _(k):
            off = base + k * CH
            pltpu.sync_copy(src_hbm.at[pl.ds(off, CH)], srcv)
            pltpu.sync_copy(dst_hbm.at[pl.ds(off, CH)], dstv)
            pltpu.async_copy(g_hbm.at[srcv], rows, sem).wait()
            pltpu.sync_copy(rows, acc_sh.at[dstv], add=True)

        plsc.subcore_barrier()
        pltpu.sync_copy(acc_sh.at[pl.ds(row0, RPS)],
                        out_hbm.at[c, pl.ds(row0, RPS)])

    return scat(g, src, dst, zeros)


def _dinv(a, b):
    return 1.0 / jnp.sqrt(a[:, :1] + b[:, :1] + 1.0)


def _tc_matmul(a, w):
    m, n = a.shape[0], w.shape[1]

    def body(a_ref, w_ref, o_ref):
        o_ref[...] = lax.dot_general(
            a_ref[...], w_ref[...], (((1,), (0,)), ((), ())),
            preferred_element_type=jnp.float32,
            precision=lax.Precision.HIGHEST)

    return pl.pallas_call(
        body, out_shape=jax.ShapeDtypeStruct((m, n), jnp.float32))(a, w)


def _tc_scale(h, c0, c1):
    def body(h_ref, a_ref, b_ref, o_ref):
        o_ref[...] = h_ref[...] * _dinv(a_ref[...], b_ref[...])

    return pl.pallas_call(
        body, out_shape=jax.ShapeDtypeStruct(h.shape, jnp.float32))(h, c0, c1)


def _tc_mid(p0, p1, g1, c0, c1, w2, b1):
    m, n = g1.shape[0], w2.shape[1]

    def body(p0_ref, p1_ref, g1_ref, a_ref, b_ref, w_ref, bias_ref, o_ref):
        dinv = _dinv(a_ref[...], b_ref[...])
        h = (p0_ref[...] + p1_ref[...] + g1_ref[...]) * dinv + bias_ref[...]
        h = jnp.maximum(h, 0.0)
        o_ref[...] = lax.dot_general(
            h, w_ref[...], (((1,), (0,)), ((), ())),
            preferred_element_type=jnp.float32,
            precision=lax.Precision.HIGHEST) * dinv

    return pl.pallas_call(
        body, out_shape=jax.ShapeDtypeStruct((m, n), jnp.float32))(
            p0, p1, g1, c0, c1, w2, b1)


def _tc_final(q0, q1, g2, c0, c1, b2):
    def body(q0_ref, q1_ref, g2_ref, a_ref, b_ref, bias_ref, o_ref):
        dinv = _dinv(a_ref[...], b_ref[...])
        z = (q0_ref[...] + q1_ref[...] + g2_ref[...]) * dinv + bias_ref[...]
        zm = z - jnp.max(z, axis=1, keepdims=True)
        o_ref[...] = zm - jnp.log(jnp.sum(jnp.exp(zm), axis=1, keepdims=True))

    return pl.pallas_call(
        body, out_shape=jax.ShapeDtypeStruct(g2.shape, jnp.float32))(
            q0, q1, g2, c0, c1, b2)


def kernel(x, edge_index, W1, b1, W2, b2):
    src = edge_index[0]
    dst = edge_index[1]
    zeros16 = jnp.zeros((N_NODES, 16), jnp.float32)
    zeros128 = jnp.zeros((N_NODES, 128), jnp.float32)
    zeros64 = jnp.zeros((N_NODES, 64), jnp.float32)
    ones16 = jnp.ones((CH, 16), jnp.float32)

    hist = _sc_degree_histogram(dst, zeros16, ones16)
    c0, c1 = hist[0], hist[1]
    hraw = _tc_matmul(x, W1)
    g1 = _tc_scale(hraw, c0, c1)
    p = _sc_scatter_rows(g1, src, dst, zeros128)
    g2 = _tc_mid(p[0], p[1], g1, c0, c1, W2, b1.reshape(1, -1))
    q = _sc_scatter_rows(g2, src, dst, zeros64)
    return _tc_final(q[0], q[1], g2, c0, c1, b2.reshape(1, -1))


# speedup vs baseline: 12.3073x; 12.3073x over previous
import functools

import jax
import jax.numpy as jnp
from jax import lax
from jax.experimental import pallas as pl
from jax.experimental.pallas import tpu as pltpu
from jax.experimental.pallas import tpu_sc as plsc

N_NODES = 10000
N_EDGES = 320000
NC = 2
NS = 16
NW = NC * NS
EPW = N_EDGES // NW
CH = 80
NCHUNK = EPW // CH
N_PAD = 10112
RPS = N_PAD // NS


def _sc_mesh():
    return plsc.VectorSubcoreMesh(core_axis_name="c", subcore_axis_name="s")


def _sc_degree_histogram(dst, zeros16, ones16):
    @functools.partial(
        pl.kernel,
        out_type=jax.ShapeDtypeStruct((NC, N_PAD, 16), jnp.float32),
        mesh=_sc_mesh(),
        scratch_types=[
            pltpu.VMEM((CH,), jnp.int32),
            pltpu.VMEM((CH, 16), jnp.float32),
            pltpu.VMEM_SHARED((N_PAD, 16), jnp.float32),
            pltpu.SemaphoreType.DMA,
        ],
    )
    def hist(dst_hbm, z_hbm, one_hbm, out_hbm, idx_v, ones_v, acc_sh, sem):
        c = lax.axis_index("c")
        s = lax.axis_index("s")
        row0 = s * RPS
        pltpu.sync_copy(one_hbm, ones_v)
        pltpu.sync_copy(z_hbm.at[pl.ds(row0, RPS)], acc_sh.at[pl.ds(row0, RPS)])
        plsc.subcore_barrier()
        base = (c * NS + s) * EPW

        @pl.loop(0, NCHUNK)
        def _(k):
            pltpu.sync_copy(dst_hbm.at[pl.ds(base + k * CH, CH)], idx_v)
            pltpu.sync_copy(ones_v, acc_sh.at[idx_v], add=True)

        plsc.subcore_barrier()
        pltpu.sync_copy(acc_sh.at[pl.ds(row0, RPS)],
                        out_hbm.at[c, pl.ds(row0, RPS)])

    return hist(dst, zeros16, ones16)


def _sc_scatter_rows(g, src, dst, zeros):
    d = g.shape[1]

    @functools.partial(
        pl.kernel,
        out_type=jax.ShapeDtypeStruct((NC, N_PAD, d), jnp.float32),
        mesh=_sc_mesh(),
        scratch_types=[
            pltpu.VMEM((CH,), jnp.int32),
            pltpu.VMEM((CH,), jnp.int32),
            pltpu.VMEM((CH, d), jnp.float32),
            pltpu.VMEM_SHARED((N_PAD, d), jnp.float32),
            pltpu.SemaphoreType.DMA,
        ],
    )
    def scat(g_hbm, src_hbm, dst_hbm, z_hbm, out_hbm, srcv, dstv, rows, acc_sh, sem):
        c = lax.axis_index("c")
        s = lax.axis_index("s")
        row0 = s * RPS
        pltpu.sync_copy(z_hbm.at[pl.ds(row0, RPS)], acc_sh.at[pl.ds(row0, RPS)])
        plsc.subcore_barrier()
        base = (c * NS + s) * EPW

        @pl.loop(0, NCHUNK)
        def _(k):
            off = base + k * CH
            pltpu.sync_copy(src_hbm.at[pl.ds(off, CH)], srcv)
            pltpu.sync_copy(dst_hbm.at[pl.ds(off, CH)], dstv)
            pltpu.async_copy(g_hbm.at[srcv], rows, sem).wait()
            pltpu.sync_copy(rows, acc_sh.at[dstv], add=True)

        plsc.subcore_barrier()
        pltpu.sync_copy(acc_sh.at[pl.ds(row0, RPS)],
                        out_hbm.at[c, pl.ds(row0, RPS)])

    return scat(g, src, dst, zeros)


def _dinv(a, b):
    return 1.0 / jnp.sqrt(a[:, :1] + b[:, :1] + 1.0)


def _tc_matmul(a, w):
    m, n = a.shape[0], w.shape[1]

    def body(a_ref, w_ref, o_ref):
        o_ref[...] = lax.dot_general(
            a_ref[...], w_ref[...], (((1,), (0,)), ((), ())),
            preferred_element_type=jnp.float32,
            precision=lax.Precision.HIGHEST)

    return pl.pallas_call(
        body, out_shape=jax.ShapeDtypeStruct((m, n), jnp.float32))(a, w)


def _tc_scale(h, c0, c1):
    def body(h_ref, a_ref, b_ref, o_ref):
        o_ref[...] = h_ref[...] * _dinv(a_ref[...], b_ref[...])

    return pl.pallas_call(
        body, out_shape=jax.ShapeDtypeStruct(h.shape, jnp.float32))(h, c0, c1)


def _tc_mid(p0, p1, g1, c0, c1, w2, b1):
    m, n = g1.shape[0], w2.shape[1]

    def body(p0_ref, p1_ref, g1_ref, a_ref, b_ref, w_ref, bias_ref, o_ref):
        dinv = _dinv(a_ref[...], b_ref[...])
        h = (p0_ref[...] + p1_ref[...] + g1_ref[...]) * dinv + bias_ref[...]
        h = jnp.maximum(h, 0.0)
        o_ref[...] = lax.dot_general(
            h, w_ref[...], (((1,), (0,)), ((), ())),
            preferred_element_type=jnp.float32,
            precision=lax.Precision.HIGHEST) * dinv

    return pl.pallas_call(
        body, out_shape=jax.ShapeDtypeStruct((m, n), jnp.float32))(
            p0, p1, g1, c0, c1, w2, b1)


def _tc_final(q0, q1, g2, c0, c1, b2):
    def body(q0_ref, q1_ref, g2_ref, a_ref, b_ref, bias_ref, o_ref):
        dinv = _dinv(a_ref[...], b_ref[...])
        z = (q0_ref[...] + q1_ref[...] + g2_ref[...]) * dinv + bias_ref[...]
        zm = z - jnp.max(z, axis=1, keepdims=True)
        o_ref[...] = zm - jnp.log(jnp.sum(jnp.exp(zm), axis=1, keepdims=True))

    return pl.pallas_call(
        body, out_shape=jax.ShapeDtypeStruct(g2.shape, jnp.float32))(
            q0, q1, g2, c0, c1, b2)


def kernel(x, edge_index, W1, b1, W2, b2):
    src = edge_index[0]
    dst = edge_index[1]
    zeros16 = jnp.zeros((N_PAD, 16), jnp.float32)
    zeros128 = jnp.zeros((N_PAD, 128), jnp.float32)
    zeros64 = jnp.zeros((N_PAD, 64), jnp.float32)
    ones16 = jnp.ones((CH, 16), jnp.float32)

    hist = _sc_degree_histogram(dst, zeros16, ones16)
    c0, c1 = hist[0, :N_NODES], hist[1, :N_NODES]
    hraw = _tc_matmul(x, W1)
    g1 = _tc_scale(hraw, c0, c1)
    p = _sc_scatter_rows(g1, src, dst, zeros128)
    W2p = jnp.pad(W2, ((0, 0), (0, 128 - W2.shape[1])))
    g2 = _tc_mid(p[0, :N_NODES], p[1, :N_NODES], g1, c0, c1, W2p, b1.reshape(1, -1))
    q = _sc_scatter_rows(g2, src, dst, zeros128)
    return _tc_final(q[0, :N_NODES, :64], q[1, :N_NODES, :64], g2[:, :64],
                     c0, c1, b2.reshape(1, -1))


# speedup vs baseline: 22.2053x; 1.8042x over previous
import functools

import jax
import jax.numpy as jnp
from jax import lax
from jax.experimental import pallas as pl
from jax.experimental.pallas import tpu as pltpu
from jax.experimental.pallas import tpu_sc as plsc

N_NODES = 10000
N_EDGES = 320000
NC = 2
NS = 16
NW = NC * NS
EPW = N_EDGES // NW
CH = 80
NCHUNK = EPW // CH
N_PAD = 10112
RPS = N_PAD // NS


def _sc_mesh():
    return plsc.VectorSubcoreMesh(core_axis_name="c", subcore_axis_name="s")


def _sc_degree_histogram(dst, zeros16, ones16):
    @functools.partial(
        pl.kernel,
        out_type=jax.ShapeDtypeStruct((NC, N_PAD, 16), jnp.float32),
        mesh=_sc_mesh(),
        scratch_types=[
            pltpu.VMEM((CH,), jnp.int32),
            pltpu.VMEM((CH,), jnp.int32),
            pltpu.VMEM((CH, 16), jnp.float32),
            pltpu.VMEM_SHARED((N_PAD, 16), jnp.float32),
            pltpu.SemaphoreType.DMA,
            pltpu.SemaphoreType.DMA,
        ],
    )
    def hist(dst_hbm, z_hbm, one_hbm, out_hbm, ia, ib, ones_v, acc_sh, sa, sb):
        c = lax.axis_index("c")
        s = lax.axis_index("s")
        w = c * NS + s
        base = w * EPW
        row0 = s * RPS
        pltpu.sync_copy(one_hbm, ones_v)
        pltpu.sync_copy(z_hbm.at[pl.ds(row0, RPS)], acc_sh.at[pl.ds(row0, RPS)])
        plsc.subcore_barrier()

        def start(k, buf, sem):
            pltpu.async_copy(dst_hbm.at[pl.ds(base + k * CH, CH)], buf, sem)

        def finish(k, buf, sem):
            pltpu.make_async_copy(dst_hbm.at[pl.ds(base + k * CH, CH)], buf, sem).wait()
            pltpu.sync_copy(ones_v, acc_sh.at[buf], add=True)

        start(0, ia, sa)

        @pl.loop(0, NCHUNK - 1, step=2)
        def _(k):
            start(k + 1, ib, sb)
            finish(k, ia, sa)
            start(k + 2, ia, sa)
            finish(k + 1, ib, sb)

        finish(NCHUNK - 1, ia, sa)

        plsc.subcore_barrier()
        pltpu.sync_copy(acc_sh.at[pl.ds(row0, RPS)],
                        out_hbm.at[c, pl.ds(row0, RPS)])

    return hist(dst, zeros16, ones16)


def _sc_scatter_rows(g, src, dst, zeros):
    d = g.shape[1]

    @functools.partial(
        pl.kernel,
        out_type=jax.ShapeDtypeStruct((NC, N_PAD, d), jnp.float32),
        mesh=_sc_mesh(),
        scratch_types=[
            pltpu.VMEM((CH,), jnp.int32),
            pltpu.VMEM((CH,), jnp.int32),
            pltpu.VMEM((CH,), jnp.int32),
            pltpu.VMEM((CH,), jnp.int32),
            pltpu.VMEM((CH, d), jnp.float32),
            pltpu.VMEM((CH, d), jnp.float32),
            pltpu.VMEM_SHARED((N_PAD, d), jnp.float32),
            pltpu.SemaphoreType.DMA,
            pltpu.SemaphoreType.DMA,
            pltpu.SemaphoreType.DMA,
            pltpu.SemaphoreType.DMA,
        ],
    )
    def scat(g_hbm, src_hbm, dst_hbm, z_hbm, out_hbm,
             sa_v, sb_v, da, db, rows_a, rows_b, acc_sh,
             sem_a, sem_b, isem_a, isem_b):
        c = lax.axis_index("c")
        s = lax.axis_index("s")
        w = c * NS + s
        base = w * EPW
        row0 = s * RPS
        pltpu.sync_copy(z_hbm.at[pl.ds(row0, RPS)], acc_sh.at[pl.ds(row0, RPS)])
        plsc.subcore_barrier()

        def start_idx(k, sbuf, dbuf, isem):
            pltpu.async_copy(src_hbm.at[pl.ds(base + k * CH, CH)], sbuf, isem)
            pltpu.async_copy(dst_hbm.at[pl.ds(base + k * CH, CH)], dbuf, isem)

        def start_gather(k, sbuf, dbuf, buf, isem, sem):
            pltpu.make_async_copy(src_hbm.at[pl.ds(base + k * CH, CH)], sbuf, isem).wait()
            pltpu.make_async_copy(dst_hbm.at[pl.ds(base + k * CH, CH)], dbuf, isem).wait()
            pltpu.async_copy(g_hbm.at[sbuf], buf, sem)

        def finish(k, dbuf, buf, sem):
            pltpu.make_async_copy(g_hbm.at[dbuf], buf, sem).wait()
            pltpu.sync_copy(buf, acc_sh.at[dbuf], add=True)

        start_idx(0, sa_v, da, isem_a)
        start_gather(0, sa_v, da, rows_a, isem_a, sem_a)

        @pl.loop(0, NCHUNK - 1, step=2)
        def _(k):
            start_idx(k + 1, sb_v, db, isem_b)
            start_gather(k + 1, sb_v, db, rows_b, isem_b, sem_b)
            finish(k, da, rows_a, sem_a)
            start_idx(k + 2, sa_v, da, isem_a)
            start_gather(k + 2, sa_v, da, rows_a, isem_a, sem_a)
            finish(k + 1, db, rows_b, sem_b)

        finish(NCHUNK - 1, da, rows_a, sem_a)

        plsc.subcore_barrier()
        pltpu.sync_copy(acc_sh.at[pl.ds(row0, RPS)],
                        out_hbm.at[c, pl.ds(row0, RPS)])

    return scat(g, src, dst, zeros)


def _dinv(a, b):
    return 1.0 / jnp.sqrt(a[:, :1] + b[:, :1] + 1.0)


def _tc_matmul(a, w):
    m, n = a.shape[0], w.shape[1]

    def body(a_ref, w_ref, o_ref):
        o_ref[...] = lax.dot_general(
            a_ref[...], w_ref[...], (((1,), (0,)), ((), ())),
            preferred_element_type=jnp.float32,
            precision=lax.Precision.HIGHEST)

    return pl.pallas_call(
        body, out_shape=jax.ShapeDtypeStruct((m, n), jnp.float32))(a, w)


def _tc_scale(h, c0, c1):
    def body(h_ref, a_ref, b_ref, o_ref):
        o_ref[...] = h_ref[...] * _dinv(a_ref[...], b_ref[...])

    return pl.pallas_call(
        body, out_shape=jax.ShapeDtypeStruct(h.shape, jnp.float32))(h, c0, c1)


def _tc_mid(p0, p1, g1, c0, c1, w2, b1):
    m, n = g1.shape[0], w2.shape[1]

    def body(p0_ref, p1_ref, g1_ref, a_ref, b_ref, w_ref, bias_ref, o_ref):
        dinv = _dinv(a_ref[...], b_ref[...])
        h = (p0_ref[...] + p1_ref[...] + g1_ref[...]) * dinv + bias_ref[...]
        h = jnp.maximum(h, 0.0)
        o_ref[...] = lax.dot_general(
            h, w_ref[...], (((1,), (0,)), ((), ())),
            preferred_element_type=jnp.float32,
            precision=lax.Precision.HIGHEST) * dinv

    return pl.pallas_call(
        body, out_shape=jax.ShapeDtypeStruct((m, n), jnp.float32))(
            p0, p1, g1, c0, c1, w2, b1)


def _tc_final(q0, q1, g2, c0, c1, b2):
    def body(q0_ref, q1_ref, g2_ref, a_ref, b_ref, bias_ref, o_ref):
        dinv = _dinv(a_ref[...], b_ref[...])
        z = (q0_ref[...] + q1_ref[...] + g2_ref[...]) * dinv + bias_ref[...]
        zm = z - jnp.max(z, axis=1, keepdims=True)
        o_ref[...] = zm - jnp.log(jnp.sum(jnp.exp(zm), axis=1, keepdims=True))

    return pl.pallas_call(
        body, out_shape=jax.ShapeDtypeStruct(g2.shape, jnp.float32))(
            q0, q1, g2, c0, c1, b2)


def kernel(x, edge_index, W1, b1, W2, b2):
    src = edge_index[0]
    dst = edge_index[1]
    zeros16 = jnp.zeros((N_PAD, 16), jnp.float32)
    zeros128 = jnp.zeros((N_PAD, 128), jnp.float32)
    zeros64 = jnp.zeros((N_PAD, 64), jnp.float32)
    ones16 = jnp.ones((CH, 16), jnp.float32)

    hist = _sc_degree_histogram(dst, zeros16, ones16)
    c0, c1 = hist[0, :N_NODES], hist[1, :N_NODES]
    hraw = _tc_matmul(x, W1)
    g1 = _tc_scale(hraw, c0, c1)
    p = _sc_scatter_rows(g1, src, dst, zeros128)
    W2p = jnp.pad(W2, ((0, 0), (0, 128 - W2.shape[1])))
    g2 = _tc_mid(p[0, :N_NODES], p[1, :N_NODES], g1, c0, c1, W2p, b1.reshape(1, -1))
    q = _sc_scatter_rows(g2, src, dst, zeros128)
    return _tc_final(q[0, :N_NODES, :64], q[1, :N_NODES, :64], g2[:, :64],
                     c0, c1, b2.reshape(1, -1))


# speedup vs baseline: 23.3349x; 1.0509x over previous
import functools

import jax
import jax.numpy as jnp
from jax import lax
from jax.experimental import pallas as pl
from jax.experimental.pallas import tpu as pltpu
from jax.experimental.pallas import tpu_sc as plsc

N_NODES = 10000
N_EDGES = 320000
NC = 2
NS = 16
NW = NC * NS
EPW = N_EDGES // NW
CH = 80
NCHUNK = EPW // CH
N_PAD = 10112
RPS = N_PAD // NS


def _sc_mesh():
    return plsc.VectorSubcoreMesh(core_axis_name="c", subcore_axis_name="s")


def _sc_degree_histogram(dst, zeros16, ones16):
    @functools.partial(
        pl.kernel,
        out_type=jax.ShapeDtypeStruct((NC, N_PAD, 16), jnp.float32),
        mesh=_sc_mesh(),
        scratch_types=[
            pltpu.VMEM((CH,), jnp.int32),
            pltpu.VMEM((CH,), jnp.int32),
            pltpu.VMEM((CH, 16), jnp.float32),
            pltpu.VMEM_SHARED((N_PAD, 16), jnp.float32),
            pltpu.SemaphoreType.DMA,
            pltpu.SemaphoreType.DMA,
        ],
    )
    def hist(dst_hbm, z_hbm, one_hbm, out_hbm, ia, ib, ones_v, acc_sh, sa, sb):
        c = lax.axis_index("c")
        s = lax.axis_index("s")
        w = c * NS + s
        base = w * EPW
        row0 = s * RPS
        pltpu.sync_copy(one_hbm, ones_v)
        pltpu.sync_copy(z_hbm.at[pl.ds(row0, RPS)], acc_sh.at[pl.ds(row0, RPS)])
        plsc.subcore_barrier()

        def start(k, buf, sem):
            pltpu.async_copy(dst_hbm.at[pl.ds(base + k * CH, CH)], buf, sem)

        def finish(k, buf, sem):
            pltpu.make_async_copy(dst_hbm.at[pl.ds(base + k * CH, CH)], buf, sem).wait()
            pltpu.sync_copy(ones_v, acc_sh.at[buf], add=True)

        start(0, ia, sa)

        @pl.loop(0, NCHUNK - 1, step=2)
        def _(k):
            start(k + 1, ib, sb)
            finish(k, ia, sa)
            start(k + 2, ia, sa)
            finish(k + 1, ib, sb)

        finish(NCHUNK - 1, ia, sa)

        plsc.subcore_barrier()
        pltpu.sync_copy(acc_sh.at[pl.ds(row0, RPS)],
                        out_hbm.at[c, pl.ds(row0, RPS)])

    return hist(dst, zeros16, ones16)


def _sc_scatter_rows(g, src, dst, zeros):
    d = g.shape[1]

    @functools.partial(
        pl.kernel,
        out_type=jax.ShapeDtypeStruct((NC, N_PAD, d), jnp.float32),
        mesh=_sc_mesh(),
        compiler_params=pltpu.CompilerParams(use_tc_tiling_on_sc=False) if d == 64 else None,
        scratch_types=[
            pltpu.VMEM((CH,), jnp.int32),
            pltpu.VMEM((CH,), jnp.int32),
            pltpu.VMEM((CH,), jnp.int32),
            pltpu.VMEM((CH,), jnp.int32),
            pltpu.VMEM((CH, d), jnp.float32),
            pltpu.VMEM((CH, d), jnp.float32),
            pltpu.VMEM_SHARED((N_PAD, d), jnp.float32),
            pltpu.SemaphoreType.DMA,
            pltpu.SemaphoreType.DMA,
            pltpu.SemaphoreType.DMA,
            pltpu.SemaphoreType.DMA,
        ],
    )
    def scat(g_hbm, src_hbm, dst_hbm, z_hbm, out_hbm,
             sa_v, sb_v, da, db, rows_a, rows_b, acc_sh,
             sem_a, sem_b, isem_a, isem_b):
        c = lax.axis_index("c")
        s = lax.axis_index("s")
        w = c * NS + s
        base = w * EPW
        row0 = s * RPS
        pltpu.sync_copy(z_hbm.at[pl.ds(row0, RPS)], acc_sh.at[pl.ds(row0, RPS)])
        plsc.subcore_barrier()

        def start_idx(k, sbuf, dbuf, isem):
            pltpu.async_copy(src_hbm.at[pl.ds(base + k * CH, CH)], sbuf, isem)
            pltpu.async_copy(dst_hbm.at[pl.ds(base + k * CH, CH)], dbuf, isem)

        def start_gather(k, sbuf, dbuf, buf, isem, sem):
            pltpu.make_async_copy(src_hbm.at[pl.ds(base + k * CH, CH)], sbuf, isem).wait()
            pltpu.make_async_copy(dst_hbm.at[pl.ds(base + k * CH, CH)], dbuf, isem).wait()
            pltpu.async_copy(g_hbm.at[sbuf], buf, sem)

        def finish(k, dbuf, buf, sem):
            pltpu.make_async_copy(g_hbm.at[dbuf], buf, sem).wait()
            pltpu.sync_copy(buf, acc_sh.at[dbuf], add=True)

        start_idx(0, sa_v, da, isem_a)
        start_gather(0, sa_v, da, rows_a, isem_a, sem_a)

        @pl.loop(0, NCHUNK - 1, step=2)
        def _(k):
            start_idx(k + 1, sb_v, db, isem_b)
            start_gather(k + 1, sb_v, db, rows_b, isem_b, sem_b)
            finish(k, da, rows_a, sem_a)
            start_idx(k + 2, sa_v, da, isem_a)
            start_gather(k + 2, sa_v, da, rows_a, isem_a, sem_a)
            finish(k + 1, db, rows_b, sem_b)

        finish(NCHUNK - 1, da, rows_a, sem_a)

        plsc.subcore_barrier()
        pltpu.sync_copy(acc_sh.at[pl.ds(row0, RPS)],
                        out_hbm.at[c, pl.ds(row0, RPS)])

    return scat(g, src, dst, zeros)


def _dinv(a, b):
    return 1.0 / jnp.sqrt(a[:, :1] + b[:, :1] + 1.0)


def _tc_matmul(a, w):
    m, n = a.shape[0], w.shape[1]

    def body(a_ref, w_ref, o_ref):
        o_ref[...] = lax.dot_general(
            a_ref[...], w_ref[...], (((1,), (0,)), ((), ())),
            preferred_element_type=jnp.float32,
            precision=lax.Precision.HIGHEST)

    return pl.pallas_call(
        body, out_shape=jax.ShapeDtypeStruct((m, n), jnp.float32))(a, w)


def _tc_scale(h, c0, c1):
    def body(h_ref, a_ref, b_ref, o_ref):
        o_ref[...] = h_ref[...] * _dinv(a_ref[...], b_ref[...])

    return pl.pallas_call(
        body, out_shape=jax.ShapeDtypeStruct(h.shape, jnp.float32))(h, c0, c1)


def _tc_mid(p0, p1, g1, c0, c1, w2, b1):
    m, n = g1.shape[0], w2.shape[1]

    def body(p0_ref, p1_ref, g1_ref, a_ref, b_ref, w_ref, bias_ref, o_ref):
        dinv = _dinv(a_ref[...], b_ref[...])
        h = (p0_ref[...] + p1_ref[...] + g1_ref[...]) * dinv + bias_ref[...]
        h = jnp.maximum(h, 0.0)
        o_ref[...] = lax.dot_general(
            h, w_ref[...], (((1,), (0,)), ((), ())),
            preferred_element_type=jnp.float32,
            precision=lax.Precision.HIGHEST) * dinv

    return pl.pallas_call(
        body, out_shape=jax.ShapeDtypeStruct((m, n), jnp.float32))(
            p0, p1, g1, c0, c1, w2, b1)


def _tc_final(q0, q1, g2, c0, c1, b2):
    def body(q0_ref, q1_ref, g2_ref, a_ref, b_ref, bias_ref, o_ref):
        dinv = _dinv(a_ref[...], b_ref[...])
        z = (q0_ref[...] + q1_ref[...] + g2_ref[...]) * dinv + bias_ref[...]
        zm = z - jnp.max(z, axis=1, keepdims=True)
        o_ref[...] = zm - jnp.log(jnp.sum(jnp.exp(zm), axis=1, keepdims=True))

    return pl.pallas_call(
        body, out_shape=jax.ShapeDtypeStruct(g2.shape, jnp.float32))(
            q0, q1, g2, c0, c1, b2)


def kernel(x, edge_index, W1, b1, W2, b2):
    src = edge_index[0]
    dst = edge_index[1]
    zeros16 = jnp.zeros((N_PAD, 16), jnp.float32)
    zeros128 = jnp.zeros((N_PAD, 128), jnp.float32)
    zeros64 = jnp.zeros((N_PAD, 64), jnp.float32)
    ones16 = jnp.ones((CH, 16), jnp.float32)

    hist = _sc_degree_histogram(dst, zeros16, ones16)
    c0, c1 = hist[0, :N_NODES], hist[1, :N_NODES]
    hraw = _tc_matmul(x, W1)
    g1 = _tc_scale(hraw, c0, c1)
    p = _sc_scatter_rows(g1, src, dst, zeros128)
    g2 = _tc_mid(p[0, :N_NODES], p[1, :N_NODES], g1, c0, c1, W2, b1.reshape(1, -1))
    q = _sc_scatter_rows(g2, src, dst, zeros64)
    return _tc_final(q[0, :N_NODES], q[1, :N_NODES], g2,
                     c0, c1, b2.reshape(1, -1))


# speedup vs baseline: 26.6128x; 1.1405x over previous
import functools

import jax
import jax.numpy as jnp
from jax import lax
from jax.experimental import pallas as pl
from jax.experimental.pallas import tpu as pltpu
from jax.experimental.pallas import tpu_sc as plsc

N_NODES = 10000
N_EDGES = 320000
NC = 2
NS = 16
NW = NC * NS
EPW = N_EDGES // NW
CH = 128
NFULL = EPW // CH
TAIL = EPW - NFULL * CH
N_PAD = 10112
RPS = N_PAD // NS


def _sc_mesh():
    return plsc.VectorSubcoreMesh(core_axis_name="c", subcore_axis_name="s")


def _sc_degree_histogram(dst, zeros16, ones16):
    @functools.partial(
        pl.kernel,
        out_type=jax.ShapeDtypeStruct((NC, N_PAD, 16), jnp.float32),
        mesh=_sc_mesh(),
        scratch_types=[
            pltpu.VMEM((CH,), jnp.int32),
            pltpu.VMEM((CH,), jnp.int32),
            pltpu.VMEM((TAIL,), jnp.int32),
            pltpu.VMEM((CH, 16), jnp.float32),
            pltpu.VMEM((TAIL, 16), jnp.float32),
            pltpu.VMEM_SHARED((N_PAD, 16), jnp.float32),
            pltpu.SemaphoreType.DMA,
            pltpu.SemaphoreType.DMA,
        ],
    )
    def hist(dst_hbm, z_hbm, one_hbm, out_hbm, ia, ib, it, ones_v, ones_t,
             acc_sh, sa, sb):
        c = lax.axis_index("c")
        s = lax.axis_index("s")
        w = c * NS + s
        base = w * EPW
        row0 = s * RPS
        pltpu.sync_copy(one_hbm, ones_v)
        pltpu.sync_copy(one_hbm.at[pl.ds(0, TAIL)], ones_t)
        pltpu.sync_copy(z_hbm.at[pl.ds(row0, RPS)], acc_sh.at[pl.ds(row0, RPS)])
        plsc.subcore_barrier()

        def start(k, buf, sem):
            pltpu.async_copy(dst_hbm.at[pl.ds(base + k * CH, CH)], buf, sem)

        def finish(k, buf, sem):
            pltpu.make_async_copy(dst_hbm.at[pl.ds(base + k * CH, CH)], buf, sem).wait()
            pltpu.sync_copy(ones_v, acc_sh.at[buf], add=True)

        start(0, ia, sa)

        @pl.loop(0, NFULL - 2, step=2)
        def _(k):
            start(k + 1, ib, sb)
            finish(k, ia, sa)
            start(k + 2, ia, sa)
            finish(k + 1, ib, sb)

        start(NFULL - 1, ib, sb)
        finish(NFULL - 2, ia, sa)
        finish(NFULL - 1, ib, sb)
        pltpu.sync_copy(dst_hbm.at[pl.ds(base + NFULL * CH, TAIL)], it)
        pltpu.sync_copy(ones_t, acc_sh.at[it], add=True)

        plsc.subcore_barrier()
        pltpu.sync_copy(acc_sh.at[pl.ds(row0, RPS)],
                        out_hbm.at[c, pl.ds(row0, RPS)])

    return hist(dst, zeros16, ones16)


def _sc_scatter_rows(g, src, dst, zeros):
    d = g.shape[1]

    @functools.partial(
        pl.kernel,
        out_type=jax.ShapeDtypeStruct((NC, N_PAD, d), jnp.float32),
        mesh=_sc_mesh(),
        compiler_params=pltpu.CompilerParams(use_tc_tiling_on_sc=False) if d == 64 else None,
        scratch_types=[
            pltpu.VMEM((CH,), jnp.int32),
            pltpu.VMEM((CH,), jnp.int32),
            pltpu.VMEM((CH,), jnp.int32),
            pltpu.VMEM((CH,), jnp.int32),
            pltpu.VMEM((TAIL,), jnp.int32),
            pltpu.VMEM((TAIL,), jnp.int32),
            pltpu.VMEM((CH, d), jnp.float32),
            pltpu.VMEM((CH, d), jnp.float32),
            pltpu.VMEM((TAIL, d), jnp.float32),
            pltpu.VMEM_SHARED((N_PAD, d), jnp.float32),
            pltpu.SemaphoreType.DMA,
            pltpu.SemaphoreType.DMA,
            pltpu.SemaphoreType.DMA,
            pltpu.SemaphoreType.DMA,
        ],
    )
    def scat(g_hbm, src_hbm, dst_hbm, z_hbm, out_hbm,
             sa_v, sb_v, da, db, st_v, dt_v, rows_a, rows_b, rows_t, acc_sh,
             sem_a, sem_b, isem_a, isem_b):
        c = lax.axis_index("c")
        s = lax.axis_index("s")
        w = c * NS + s
        base = w * EPW
        row0 = s * RPS
        pltpu.sync_copy(z_hbm.at[pl.ds(row0, RPS)], acc_sh.at[pl.ds(row0, RPS)])
        plsc.subcore_barrier()

        def start_idx(k, sbuf, dbuf, isem):
            pltpu.async_copy(src_hbm.at[pl.ds(base + k * CH, CH)], sbuf, isem)
            pltpu.async_copy(dst_hbm.at[pl.ds(base + k * CH, CH)], dbuf, isem)

        def start_gather(k, sbuf, dbuf, buf, isem, sem):
            pltpu.make_async_copy(src_hbm.at[pl.ds(base + k * CH, CH)], sbuf, isem).wait()
            pltpu.make_async_copy(dst_hbm.at[pl.ds(base + k * CH, CH)], dbuf, isem).wait()
            pltpu.async_copy(g_hbm.at[sbuf], buf, sem)

        def finish(k, dbuf, buf, sem):
            pltpu.make_async_copy(g_hbm.at[dbuf], buf, sem).wait()
            pltpu.sync_copy(buf, acc_sh.at[dbuf], add=True)

        start_idx(0, sa_v, da, isem_a)
        start_gather(0, sa_v, da, rows_a, isem_a, sem_a)

        @pl.loop(0, NFULL - 2, step=2)
        def _(k):
            start_idx(k + 1, sb_v, db, isem_b)
            start_gather(k + 1, sb_v, db, rows_b, isem_b, sem_b)
            finish(k, da, rows_a, sem_a)
            start_idx(k + 2, sa_v, da, isem_a)
            start_gather(k + 2, sa_v, da, rows_a, isem_a, sem_a)
            finish(k + 1, db, rows_b, sem_b)

        start_idx(NFULL - 1, sb_v, db, isem_b)
        start_gather(NFULL - 1, sb_v, db, rows_b, isem_b, sem_b)
        finish(NFULL - 2, da, rows_a, sem_a)
        finish(NFULL - 1, db, rows_b, sem_b)
        tb = base + NFULL * CH
        pltpu.sync_copy(src_hbm.at[pl.ds(tb, TAIL)], st_v)
        pltpu.sync_copy(dst_hbm.at[pl.ds(tb, TAIL)], dt_v)
        pltpu.async_copy(g_hbm.at[st_v], rows_t, sem_a).wait()
        pltpu.sync_copy(rows_t, acc_sh.at[dt_v], add=True)

        plsc.subcore_barrier()
        pltpu.sync_copy(acc_sh.at[pl.ds(row0, RPS)],
                        out_hbm.at[c, pl.ds(row0, RPS)])

    return scat(g, src, dst, zeros)


def _dinv(a, b):
    return 1.0 / jnp.sqrt(a[:, :1] + b[:, :1] + 1.0)


def _tc_matmul(a, w):
    m, n = a.shape[0], w.shape[1]

    def body(a_ref, w_ref, o_ref):
        o_ref[...] = lax.dot_general(
            a_ref[...], w_ref[...], (((1,), (0,)), ((), ())),
            preferred_element_type=jnp.float32,
            precision=lax.Precision.HIGHEST)

    return pl.pallas_call(
        body, out_shape=jax.ShapeDtypeStruct((m, n), jnp.float32))(a, w)


def _tc_scale(h, c0, c1):
    def body(h_ref, a_ref, b_ref, o_ref):
        o_ref[...] = h_ref[...] * _dinv(a_ref[...], b_ref[...])

    return pl.pallas_call(
        body, out_shape=jax.ShapeDtypeStruct(h.shape, jnp.float32))(h, c0, c1)


def _tc_mid(p0, p1, g1, c0, c1, w2, b1):
    m, n = g1.shape[0], w2.shape[1]

    def body(p0_ref, p1_ref, g1_ref, a_ref, b_ref, w_ref, bias_ref, o_ref):
        dinv = _dinv(a_ref[...], b_ref[...])
        h = (p0_ref[...] + p1_ref[...] + g1_ref[...]) * dinv + bias_ref[...]
        h = jnp.maximum(h, 0.0)
        o_ref[...] = lax.dot_general(
            h, w_ref[...], (((1,), (0,)), ((), ())),
            preferred_element_type=jnp.float32,
            precision=lax.Precision.HIGHEST) * dinv

    return pl.pallas_call(
        body, out_shape=jax.ShapeDtypeStruct((m, n), jnp.float32))(
            p0, p1, g1, c0, c1, w2, b1)


def _tc_final(q0, q1, g2, c0, c1, b2):
    def body(q0_ref, q1_ref, g2_ref, a_ref, b_ref, bias_ref, o_ref):
        dinv = _dinv(a_ref[...], b_ref[...])
        z = (q0_ref[...] + q1_ref[...] + g2_ref[...]) * dinv + bias_ref[...]
        zm = z - jnp.max(z, axis=1, keepdims=True)
        o_ref[...] = zm - jnp.log(jnp.sum(jnp.exp(zm), axis=1, keepdims=True))

    return pl.pallas_call(
        body, out_shape=jax.ShapeDtypeStruct(g2.shape, jnp.float32))(
            q0, q1, g2, c0, c1, b2)


def kernel(x, edge_index, W1, b1, W2, b2):
    src = edge_index[0]
    dst = edge_index[1]
    zeros16 = jnp.zeros((N_PAD, 16), jnp.float32)
    zeros128 = jnp.zeros((N_PAD, 128), jnp.float32)
    zeros64 = jnp.zeros((N_PAD, 64), jnp.float32)
    ones16 = jnp.ones((CH, 16), jnp.float32)

    hist = _sc_degree_histogram(dst, zeros16, ones16)
    c0, c1 = hist[0, :N_NODES], hist[1, :N_NODES]
    hraw = _tc_matmul(x, W1)
    g1 = _tc_scale(hraw, c0, c1)
    p = _sc_scatter_rows(g1, src, dst, zeros128)
    g2 = _tc_mid(p[0, :N_NODES], p[1, :N_NODES], g1, c0, c1, W2, b1.reshape(1, -1))
    q = _sc_scatter_rows(g2, src, dst, zeros64)
    return _tc_final(q[0, :N_NODES], q[1, :N_NODES], g2,
                     c0, c1, b2.reshape(1, -1))
